# baseline (speedup 1.0000x reference)
import functools

import jax
import jax.numpy as jnp
from jax import lax
from jax.experimental import pallas as pl
from jax.experimental.pallas import tpu as pltpu
from jax.experimental.pallas import tpu_sc as plsc

N = 200000
B = 32
D = 256

NC = 2
NS = 16
NW = NC * NS

RPW0 = N // NW
NIDS = 6256
NVEC = NIDS // 16
RB = 384

_mesh = plsc.VectorSubcoreMesh(core_axis_name="c", subcore_axis_name="s")


@functools.partial(
    pl.kernel,
    out_type=jax.ShapeDtypeStruct((N, D), jnp.float32),
    mesh=_mesh,
    scratch_types=[
        pltpu.VMEM((NIDS,), jnp.int32),
        pltpu.VMEM((B, D), jnp.float32),
        pltpu.VMEM((RB, D), jnp.float32),
        pltpu.VMEM((8, D), jnp.float32),
        pltpu.SemaphoreType.DMA,
    ],
)
def _broadcast_sc(ids_hbm, glob_hbm, out_hbm, idx_v, glob_v, buf, bblk, sem):
    wid = lax.axis_index("s") * NC + lax.axis_index("c")

    base = wid * RPW0 - ((2 * wid) & 7)
    nbase = (wid + 1) * RPW0 - ((2 * wid + 2) & 7)
    rpw = nbase - base

    pltpu.sync_copy(ids_hbm.at[pl.ds(pl.multiple_of(base, 8), NIDS)], idx_v)
    pltpu.sync_copy(glob_hbm, glob_v)

    def load_vec(m):
        return idx_v[pl.ds(pl.multiple_of(m * 16, 16), 16)]

    def lower_bound(t):
        tv = jnp.broadcast_to(t, (16,))
        pos = jnp.int32(0)
        for step in (256, 128, 64, 32, 16, 8, 4, 2, 1):
            cand = pos + step
            v = load_vec(jnp.minimum(cand - 1, NVEC - 1))
            ok = jnp.logical_and(cand <= NVEC, v[15] < t)
            pos = jnp.where(ok, cand, pos)
        v = load_vec(jnp.minimum(pos, NVEC - 1))
        w = jnp.where(v < tv, jnp.ones((16,), jnp.int32),
                      jnp.zeros((16,), jnp.int32))
        pc = w[0]
        for l in range(1, 16):
            pc = pc + w[l]
        return jnp.minimum(pos * 16 + pc, rpw)

    def emit(b, prev):
        nxt = lower_bound(b + 1)
        cnt = nxt - prev
        gs = base + prev
        ge = base + nxt
        iu = (gs + 7) & (-8)
        idn = ge & (-8)

        @pl.when(cnt > 0)
        def _():
            @pl.when(idn > iu)
            def _():
                def fill_row(r, c2):
                    for c in range(D // 16):
                        buf[r, pl.ds(c * 16, 16)] = glob_v[b, pl.ds(c * 16, 16)]
                    return c2

                lax.fori_loop(0, RB, fill_row, 0)

                size = idn - iu
                nfull = size // RB

                def dma_full(i, o):
                    pltpu.async_copy(
                        buf.at[pl.ds(0, RB)],
                        out_hbm.at[pl.ds(pl.multiple_of(o, 8), RB)],
                        sem)
                    return o + RB

                o = lax.fori_loop(0, nfull, dma_full, iu)

                for sz in (256, 128, 64, 32, 16, 8):
                    @pl.when((size & sz) != 0)
                    def _(sz=sz, o=o):
                        pltpu.async_copy(
                            buf.at[pl.ds(0, sz)],
                            out_hbm.at[pl.ds(pl.multiple_of(o, 8), sz)],
                            sem)
                    o = o + (size & sz)

                def drain(i, c2):
                    pltpu.make_async_copy(
                        buf.at[pl.ds(0, RB)],
                        out_hbm.at[pl.ds(pl.multiple_of(iu, 8), RB)],
                        sem).wait()
                    return c2

                lax.fori_loop(0, nfull, drain, 0)
                for sz in (256, 128, 64, 32, 16, 8):
                    @pl.when((size & sz) != 0)
                    def _(sz=sz):
                        pltpu.make_async_copy(
                            buf.at[pl.ds(0, sz)],
                            out_hbm.at[pl.ds(pl.multiple_of(iu, 8), sz)],
                            sem).wait()

            @pl.when((ge & 7) != 0)
            def _():
                p0 = idn - base
                vb = load_vec(p0 >> 4)
                sel_hi = (p0 & 15) == 8
                for r8 in range(8):
                    idr = jnp.where(sel_hi, vb[8 + r8], vb[r8])
                    for c in range(D // 16):
                        bblk[r8, pl.ds(c * 16, 16)] = glob_v[idr,
                                                             pl.ds(c * 16, 16)]
                pltpu.sync_copy(
                    bblk, out_hbm.at[pl.ds(pl.multiple_of(idn, 8), 8)])

        return nxt

    lax.fori_loop(0, B, emit, jnp.int32(0))


def kernel(x, glob, batch_ids):
    ids = batch_ids.astype(jnp.int32)
    return _broadcast_sc(ids, glob)


# speedup vs baseline: 1.1200x; 1.1200x over previous
import functools

import jax
import jax.numpy as jnp
from jax import lax
from jax.experimental import pallas as pl
from jax.experimental.pallas import tpu as pltpu
from jax.experimental.pallas import tpu_sc as plsc

N = 200000
B = 32
D = 256

NC = 2
NS = 16
NW = NC * NS

RPW0 = N // NW
NIDS = 6256
NVEC = NIDS // 16
RB = 256

_mesh = plsc.VectorSubcoreMesh(core_axis_name="c", subcore_axis_name="s")


@functools.partial(
    pl.kernel,
    out_type=jax.ShapeDtypeStruct((N, D), jnp.float32),
    mesh=_mesh,
    scratch_types=[
        pltpu.VMEM((NIDS,), jnp.int32),
        pltpu.VMEM((B, D), jnp.float32),
        pltpu.VMEM((RB, D), jnp.float32),
        pltpu.VMEM((8, D), jnp.float32),
        pltpu.SemaphoreType.DMA,
    ],
)
def _broadcast_sc(ids_hbm, glob_hbm, out_hbm, idx_v, glob_v, buf, bblk, sem):
    wid = lax.axis_index("s") * NC + lax.axis_index("c")

    base = wid * RPW0 - ((2 * wid) & 7)
    nbase = (wid + 1) * RPW0 - ((2 * wid + 2) & 7)
    rpw = nbase - base

    pltpu.sync_copy(ids_hbm.at[pl.ds(pl.multiple_of(base, 8), NIDS)], idx_v)
    pltpu.sync_copy(glob_hbm, glob_v)

    def load_vec(m):
        return idx_v[pl.ds(pl.multiple_of(m * 16, 16), 16)]

    def lower_bound(t):
        tv = jnp.broadcast_to(t, (16,))
        pos = jnp.int32(0)
        for step in (256, 128, 64, 32, 16, 8, 4, 2, 1):
            cand = pos + step
            v = load_vec(jnp.minimum(cand - 1, NVEC - 1))
            ok = jnp.logical_and(cand <= NVEC, v[15] < t)
            pos = jnp.where(ok, cand, pos)
        v = load_vec(jnp.minimum(pos, NVEC - 1))
        w = jnp.where(v < tv, jnp.ones((16,), jnp.int32),
                      jnp.zeros((16,), jnp.int32))
        pc = w[0]
        for l in range(1, 16):
            pc = pc + w[l]
        return jnp.minimum(pos * 16 + pc, rpw)

    def emit(b, prev):
        nxt = lower_bound(b + 1)
        cnt = nxt - prev
        gs = base + prev
        ge = base + nxt
        iu = (gs + 7) & (-8)
        idn = ge & (-8)

        @pl.when(cnt > 0)
        def _():
            @pl.when(idn > iu)
            def _():
                def fill_row(r, c2):
                    for c in range(D // 16):
                        buf[r, pl.ds(c * 16, 16)] = glob_v[b, pl.ds(c * 16, 16)]
                    return c2

                lax.fori_loop(0, RB, fill_row, 0)

                size = idn - iu
                nfull = size // RB

                def dma_full(i, o):
                    pltpu.async_copy(
                        buf.at[pl.ds(0, RB)],
                        out_hbm.at[pl.ds(pl.multiple_of(o, 8), RB)],
                        sem)
                    return o + RB

                o = lax.fori_loop(0, nfull, dma_full, iu)

                for sz in (256, 128, 64, 32, 16, 8):
                    @pl.when((size & sz) != 0)
                    def _(sz=sz, o=o):
                        pltpu.async_copy(
                            buf.at[pl.ds(0, sz)],
                            out_hbm.at[pl.ds(pl.multiple_of(o, 8), sz)],
                            sem)
                    o = o + (size & sz)

                def drain(i, c2):
                    pltpu.make_async_copy(
                        buf.at[pl.ds(0, RB)],
                        out_hbm.at[pl.ds(pl.multiple_of(iu, 8), RB)],
                        sem).wait()
                    return c2

                lax.fori_loop(0, nfull, drain, 0)
                for sz in (256, 128, 64, 32, 16, 8):
                    @pl.when((size & sz) != 0)
                    def _(sz=sz):
                        pltpu.make_async_copy(
                            buf.at[pl.ds(0, sz)],
                            out_hbm.at[pl.ds(pl.multiple_of(iu, 8), sz)],
                            sem).wait()

            @pl.when((ge & 7) != 0)
            def _():
                p0 = idn - base
                vb = load_vec(p0 >> 4)
                sel_hi = (p0 & 15) == 8
                for r8 in range(8):
                    idr = jnp.where(sel_hi, vb[8 + r8], vb[r8])
                    for c in range(D // 16):
                        bblk[r8, pl.ds(c * 16, 16)] = glob_v[idr,
                                                             pl.ds(c * 16, 16)]
                pltpu.sync_copy(
                    bblk, out_hbm.at[pl.ds(pl.multiple_of(idn, 8), 8)])

        return nxt

    lax.fori_loop(0, B, emit, jnp.int32(0))


def kernel(x, glob, batch_ids):
    ids = batch_ids.astype(jnp.int32)
    return _broadcast_sc(ids, glob)


# speedup vs baseline: 1.1470x; 1.0241x over previous
import functools

import jax
import jax.numpy as jnp
from jax import lax
from jax.experimental import pallas as pl
from jax.experimental.pallas import tpu as pltpu
from jax.experimental.pallas import tpu_sc as plsc

N = 200000
B = 32
D = 256

NC = 2
NS = 16
NW = NC * NS

RPW0 = N // NW
NIDS = 6256
NVEC = NIDS // 16
RB = 256

_mesh = plsc.VectorSubcoreMesh(core_axis_name="c", subcore_axis_name="s")


@functools.partial(
    pl.kernel,
    out_type=jax.ShapeDtypeStruct((N, D), jnp.float32),
    mesh=_mesh,
    scratch_types=[
        pltpu.VMEM((NIDS,), jnp.int32),
        pltpu.VMEM((B, D), jnp.float32),
        pltpu.VMEM((RB, D), jnp.float32),
        pltpu.VMEM((8, D), jnp.float32),
        pltpu.SemaphoreType.DMA,
    ],
)
def _broadcast_sc(ids_hbm, glob_hbm, out_hbm, idx_v, glob_v, buf, bblk, sem):
    wid = lax.axis_index("s") * NC + lax.axis_index("c")

    base = wid * RPW0 - ((2 * wid) & 7)
    nbase = (wid + 1) * RPW0 - ((2 * wid + 2) & 7)
    rpw = nbase - base

    pltpu.sync_copy(ids_hbm.at[pl.ds(pl.multiple_of(base, 8), NIDS)], idx_v)
    pltpu.sync_copy(glob_hbm, glob_v)

    def load_vec(m):
        return idx_v[pl.ds(pl.multiple_of(m * 16, 16), 16)]

    def lower_bound(t):
        tv = jnp.broadcast_to(t, (16,))
        pos = jnp.int32(0)
        for step in (256, 128, 64, 32, 16, 8, 4, 2, 1):
            cand = pos + step
            v = load_vec(jnp.minimum(cand - 1, NVEC - 1))
            ok = jnp.logical_and(cand <= NVEC, v[15] < t)
            pos = jnp.where(ok, cand, pos)
        v = load_vec(jnp.minimum(pos, NVEC - 1))
        w = jnp.where(v < tv, jnp.ones((16,), jnp.int32),
                      jnp.zeros((16,), jnp.int32))
        pc = w[0]
        for l in range(1, 16):
            pc = pc + w[l]
        return jnp.minimum(pos * 16 + pc, rpw)

    def emit(b, prev):
        nxt = lower_bound(b + 1)
        cnt = nxt - prev
        gs = base + prev
        ge = base + nxt
        iu = (gs + 7) & (-8)
        idn = ge & (-8)

        @pl.when(cnt > 0)
        def _():
            @pl.when(idn > iu)
            def _():
                def fill_row(r, c2):
                    for c in range(D // 16):
                        buf[r, pl.ds(c * 16, 16)] = glob_v[b, pl.ds(c * 16, 16)]
                    return c2

                lax.fori_loop(0, RB, fill_row, 0)

                size = idn - iu
                nfull = size // RB

                def dma_full(i, o):
                    pltpu.async_copy(
                        buf.at[pl.ds(0, RB)],
                        out_hbm.at[pl.ds(pl.multiple_of(o, 8), RB)],
                        sem)
                    return o + RB

                o = lax.fori_loop(0, nfull, dma_full, iu)

                for sz in (128, 64, 32, 16, 8):
                    @pl.when((size & sz) != 0)
                    def _(sz=sz, o=o):
                        pltpu.sync_copy(
                            buf.at[pl.ds(0, sz)],
                            out_hbm.at[pl.ds(pl.multiple_of(o, 8), sz)])
                    o = o + (size & sz)

                def drain(i, c2):
                    pltpu.make_async_copy(
                        buf.at[pl.ds(0, RB)],
                        out_hbm.at[pl.ds(pl.multiple_of(iu, 8), RB)],
                        sem).wait()
                    return c2

                lax.fori_loop(0, nfull, drain, 0)

            @pl.when((ge & 7) != 0)
            def _():
                p0 = idn - base
                vb = load_vec(p0 >> 4)
                sel_hi = (p0 & 15) == 8
                for r8 in range(8):
                    idr = jnp.where(sel_hi, vb[8 + r8], vb[r8])
                    for c in range(D // 16):
                        bblk[r8, pl.ds(c * 16, 16)] = glob_v[idr,
                                                             pl.ds(c * 16, 16)]
                pltpu.sync_copy(
                    bblk, out_hbm.at[pl.ds(pl.multiple_of(idn, 8), 8)])

        return nxt

    lax.fori_loop(0, B, emit, jnp.int32(0))


def kernel(x, glob, batch_ids):
    ids = batch_ids.astype(jnp.int32)
    return _broadcast_sc(ids, glob)


# speedup vs baseline: 1.2741x; 1.1108x over previous
import functools

import jax
import jax.numpy as jnp
from jax import lax
from jax.experimental import pallas as pl
from jax.experimental.pallas import tpu as pltpu
from jax.experimental.pallas import tpu_sc as plsc

N = 200000
B = 32
D = 256

NC = 2
NS = 16
NW = NC * NS

RPW0 = N // NW
NIDS = 6256
NVEC = NIDS // 16
RB = 128

_mesh = plsc.VectorSubcoreMesh(core_axis_name="c", subcore_axis_name="s")


@functools.partial(
    pl.kernel,
    out_type=jax.ShapeDtypeStruct((N, D), jnp.float32),
    mesh=_mesh,
    scratch_types=[
        pltpu.VMEM((NIDS,), jnp.int32),
        pltpu.VMEM((B, D), jnp.float32),
        pltpu.VMEM((RB, D), jnp.float32),
        pltpu.VMEM((8, D), jnp.float32),
        pltpu.SemaphoreType.DMA,
    ],
)
def _broadcast_sc(ids_hbm, glob_hbm, out_hbm, idx_v, glob_v, buf, bblk, sem):
    wid = lax.axis_index("s") * NC + lax.axis_index("c")

    base = wid * RPW0 - ((2 * wid) & 7)
    nbase = (wid + 1) * RPW0 - ((2 * wid + 2) & 7)
    rpw = nbase - base

    pltpu.sync_copy(ids_hbm.at[pl.ds(pl.multiple_of(base, 8), NIDS)], idx_v)
    pltpu.sync_copy(glob_hbm, glob_v)

    def load_vec(m):
        return idx_v[pl.ds(pl.multiple_of(m * 16, 16), 16)]

    def lower_bound(t):
        tv = jnp.broadcast_to(t, (16,))
        pos = jnp.int32(0)
        for step in (256, 128, 64, 32, 16, 8, 4, 2, 1):
            cand = pos + step
            v = load_vec(jnp.minimum(cand - 1, NVEC - 1))
            ok = jnp.logical_and(cand <= NVEC, v[15] < t)
            pos = jnp.where(ok, cand, pos)
        v = load_vec(jnp.minimum(pos, NVEC - 1))
        w = jnp.where(v < tv, jnp.ones((16,), jnp.int32),
                      jnp.zeros((16,), jnp.int32))
        pc = w[0]
        for l in range(1, 16):
            pc = pc + w[l]
        return jnp.minimum(pos * 16 + pc, rpw)

    def emit(b, prev):
        nxt = lower_bound(b + 1)
        cnt = nxt - prev
        gs = base + prev
        ge = base + nxt
        iu = (gs + 7) & (-8)
        idn = ge & (-8)

        @pl.when(cnt > 0)
        def _():
            @pl.when(idn > iu)
            def _():
                def fill_row(r, c2):
                    for c in range(D // 16):
                        buf[r, pl.ds(c * 16, 16)] = glob_v[b, pl.ds(c * 16, 16)]
                    return c2

                lax.fori_loop(0, RB, fill_row, 0)

                size = idn - iu
                nfull = size // RB

                def dma_full(i, o):
                    pltpu.async_copy(
                        buf.at[pl.ds(0, RB)],
                        out_hbm.at[pl.ds(pl.multiple_of(o, 8), RB)],
                        sem)
                    return o + RB

                o = lax.fori_loop(0, nfull, dma_full, iu)

                for sz in (128, 64, 32, 16, 8):
                    @pl.when((size & sz) != 0)
                    def _(sz=sz, o=o):
                        pltpu.sync_copy(
                            buf.at[pl.ds(0, sz)],
                            out_hbm.at[pl.ds(pl.multiple_of(o, 8), sz)])
                    o = o + (size & sz)

                def drain(i, c2):
                    pltpu.make_async_copy(
                        buf.at[pl.ds(0, RB)],
                        out_hbm.at[pl.ds(pl.multiple_of(iu, 8), RB)],
                        sem).wait()
                    return c2

                lax.fori_loop(0, nfull, drain, 0)

            @pl.when((ge & 7) != 0)
            def _():
                p0 = idn - base
                vb = load_vec(p0 >> 4)
                sel_hi = (p0 & 15) == 8
                for r8 in range(8):
                    idr = jnp.where(sel_hi, vb[8 + r8], vb[r8])
                    for c in range(D // 16):
                        bblk[r8, pl.ds(c * 16, 16)] = glob_v[idr,
                                                             pl.ds(c * 16, 16)]
                pltpu.sync_copy(
                    bblk, out_hbm.at[pl.ds(pl.multiple_of(idn, 8), 8)])

        return nxt

    lax.fori_loop(0, B, emit, jnp.int32(0))


def kernel(x, glob, batch_ids):
    ids = batch_ids.astype(jnp.int32)
    return _broadcast_sc(ids, glob)


# speedup vs baseline: 1.3592x; 1.0668x over previous
import functools

import jax
import jax.numpy as jnp
from jax import lax
from jax.experimental import pallas as pl
from jax.experimental.pallas import tpu as pltpu
from jax.experimental.pallas import tpu_sc as plsc

N = 200000
B = 32
D = 256

NC = 2
NS = 16
NW = NC * NS

RPW0 = N // NW
NIDS = 6256
NVEC = NIDS // 16
RB = 64

_mesh = plsc.VectorSubcoreMesh(core_axis_name="c", subcore_axis_name="s")


@functools.partial(
    pl.kernel,
    out_type=jax.ShapeDtypeStruct((N, D), jnp.float32),
    mesh=_mesh,
    scratch_types=[
        pltpu.VMEM((NIDS,), jnp.int32),
        pltpu.VMEM((B, D), jnp.float32),
        pltpu.VMEM((RB, D), jnp.float32),
        pltpu.VMEM((8, D), jnp.float32),
        pltpu.SemaphoreType.DMA,
    ],
)
def _broadcast_sc(ids_hbm, glob_hbm, out_hbm, idx_v, glob_v, buf, bblk, sem):
    wid = lax.axis_index("s") * NC + lax.axis_index("c")

    base = wid * RPW0 - ((2 * wid) & 7)
    nbase = (wid + 1) * RPW0 - ((2 * wid + 2) & 7)
    rpw = nbase - base

    pltpu.sync_copy(ids_hbm.at[pl.ds(pl.multiple_of(base, 8), NIDS)], idx_v)
    pltpu.sync_copy(glob_hbm, glob_v)

    def load_vec(m):
        return idx_v[pl.ds(pl.multiple_of(m * 16, 16), 16)]

    def lower_bound(t):
        tv = jnp.broadcast_to(t, (16,))
        pos = jnp.int32(0)
        for step in (256, 128, 64, 32, 16, 8, 4, 2, 1):
            cand = pos + step
            v = load_vec(jnp.minimum(cand - 1, NVEC - 1))
            ok = jnp.logical_and(cand <= NVEC, v[15] < t)
            pos = jnp.where(ok, cand, pos)
        v = load_vec(jnp.minimum(pos, NVEC - 1))
        w = jnp.where(v < tv, jnp.ones((16,), jnp.int32),
                      jnp.zeros((16,), jnp.int32))
        pc = w[0]
        for l in range(1, 16):
            pc = pc + w[l]
        return jnp.minimum(pos * 16 + pc, rpw)

    def emit(b, prev):
        nxt = lower_bound(b + 1)
        cnt = nxt - prev
        gs = base + prev
        ge = base + nxt
        iu = (gs + 7) & (-8)
        idn = ge & (-8)

        @pl.when(cnt > 0)
        def _():
            @pl.when(idn > iu)
            def _():
                def fill_row(r, c2):
                    for c in range(D // 16):
                        buf[r, pl.ds(c * 16, 16)] = glob_v[b, pl.ds(c * 16, 16)]
                    return c2

                lax.fori_loop(0, RB, fill_row, 0)

                size = idn - iu
                nfull = size // RB

                def dma_full(i, o):
                    pltpu.async_copy(
                        buf.at[pl.ds(0, RB)],
                        out_hbm.at[pl.ds(pl.multiple_of(o, 8), RB)],
                        sem)
                    return o + RB

                o = lax.fori_loop(0, nfull, dma_full, iu)

                for sz in (128, 64, 32, 16, 8):
                    @pl.when((size & sz) != 0)
                    def _(sz=sz, o=o):
                        pltpu.sync_copy(
                            buf.at[pl.ds(0, sz)],
                            out_hbm.at[pl.ds(pl.multiple_of(o, 8), sz)])
                    o = o + (size & sz)

                def drain(i, c2):
                    pltpu.make_async_copy(
                        buf.at[pl.ds(0, RB)],
                        out_hbm.at[pl.ds(pl.multiple_of(iu, 8), RB)],
                        sem).wait()
                    return c2

                lax.fori_loop(0, nfull, drain, 0)

            @pl.when((ge & 7) != 0)
            def _():
                p0 = idn - base
                vb = load_vec(p0 >> 4)
                sel_hi = (p0 & 15) == 8
                for r8 in range(8):
                    idr = jnp.where(sel_hi, vb[8 + r8], vb[r8])
                    for c in range(D // 16):
                        bblk[r8, pl.ds(c * 16, 16)] = glob_v[idr,
                                                             pl.ds(c * 16, 16)]
                pltpu.sync_copy(
                    bblk, out_hbm.at[pl.ds(pl.multiple_of(idn, 8), 8)])

        return nxt

    lax.fori_loop(0, B, emit, jnp.int32(0))


def kernel(x, glob, batch_ids):
    ids = batch_ids.astype(jnp.int32)
    return _broadcast_sc(ids, glob)


# speedup vs baseline: 1.3881x; 1.0212x over previous
import functools

import jax
import jax.numpy as jnp
from jax import lax
from jax.experimental import pallas as pl
from jax.experimental.pallas import tpu as pltpu
from jax.experimental.pallas import tpu_sc as plsc

N = 200000
B = 32
D = 256

NC = 2
NS = 16
NW = NC * NS

RPW0 = N // NW
NIDS = 6256
NVEC = NIDS // 16
RB = 64
NQ = 24

_mesh = plsc.VectorSubcoreMesh(core_axis_name="c", subcore_axis_name="s")


@functools.partial(
    pl.kernel,
    out_type=jax.ShapeDtypeStruct((N, D), jnp.float32),
    mesh=_mesh,
    scratch_types=[
        pltpu.VMEM((NIDS,), jnp.int32),
        pltpu.VMEM((B, D), jnp.float32),
        pltpu.VMEM((RB, D), jnp.float32),
        pltpu.VMEM((8, D), jnp.float32),
        pltpu.SemaphoreType.DMA,
    ],
)
def _broadcast_sc(ids_hbm, glob_hbm, out_hbm, idx_v, glob_v, buf, bblk, sem):
    wid = lax.axis_index("s") * NC + lax.axis_index("c")

    base = wid * RPW0 - ((2 * wid) & 7)
    nbase = (wid + 1) * RPW0 - ((2 * wid + 2) & 7)
    rpw = nbase - base

    pltpu.sync_copy(ids_hbm.at[pl.ds(pl.multiple_of(base, 8), NIDS)], idx_v)
    pltpu.sync_copy(glob_hbm, glob_v)

    def load_vec(m):
        return idx_v[pl.ds(pl.multiple_of(m * 16, 16), 16)]

    def lower_bound(t):
        tv = jnp.broadcast_to(t, (16,))
        pos = jnp.int32(0)
        for step in (256, 128, 64, 32, 16, 8, 4, 2, 1):
            cand = pos + step
            v = load_vec(jnp.minimum(cand - 1, NVEC - 1))
            ok = jnp.logical_and(cand <= NVEC, v[15] < t)
            pos = jnp.where(ok, cand, pos)
        v = load_vec(jnp.minimum(pos, NVEC - 1))
        w = jnp.where(v < tv, jnp.ones((16,), jnp.int32),
                      jnp.zeros((16,), jnp.int32))
        pc = w[0]
        for l in range(1, 16):
            pc = pc + w[l]
        return jnp.minimum(pos * 16 + pc, rpw)

    def emit(b, prev):
        nxt = lower_bound(b + 1)
        cnt = nxt - prev
        gs = base + prev
        ge = base + nxt
        iu = (gs + 7) & (-8)
        idn = ge & (-8)

        @pl.when(cnt > 0)
        def _():
            @pl.when(idn > iu)
            def _():
                def fill_row(r, c2):
                    for c in range(D // 16):
                        buf[r, pl.ds(c * 16, 16)] = glob_v[b, pl.ds(c * 16, 16)]
                    return c2

                lax.fori_loop(0, RB, fill_row, 0)

                size = idn - iu
                nfull = size // RB

                def dma_full(i, o):
                    pltpu.async_copy(
                        buf.at[pl.ds(0, RB)],
                        out_hbm.at[pl.ds(pl.multiple_of(o, 8), RB)],
                        sem)

                    @pl.when(i >= NQ)
                    def _():
                        pltpu.make_async_copy(
                            buf.at[pl.ds(0, RB)],
                            out_hbm.at[pl.ds(pl.multiple_of(iu, 8), RB)],
                            sem).wait()

                    return o + RB

                o = lax.fori_loop(0, nfull, dma_full, iu)

                for sz in [x for x in (128, 64, 32, 16, 8) if x < RB]:
                    @pl.when((size & sz) != 0)
                    def _(sz=sz, o=o):
                        pltpu.sync_copy(
                            buf.at[pl.ds(0, sz)],
                            out_hbm.at[pl.ds(pl.multiple_of(o, 8), sz)])
                    o = o + (size & sz)

                def drain(i, c2):
                    pltpu.make_async_copy(
                        buf.at[pl.ds(0, RB)],
                        out_hbm.at[pl.ds(pl.multiple_of(iu, 8), RB)],
                        sem).wait()
                    return c2

                lax.fori_loop(0, jnp.minimum(nfull, NQ), drain, 0)

            @pl.when((ge & 7) != 0)
            def _():
                p0 = idn - base
                vb = load_vec(p0 >> 4)
                sel_hi = (p0 & 15) == 8
                for r8 in range(8):
                    idr = jnp.where(sel_hi, vb[8 + r8], vb[r8])
                    for c in range(D // 16):
                        bblk[r8, pl.ds(c * 16, 16)] = glob_v[idr,
                                                             pl.ds(c * 16, 16)]
                pltpu.sync_copy(
                    bblk, out_hbm.at[pl.ds(pl.multiple_of(idn, 8), 8)])

        return nxt

    lax.fori_loop(0, B, emit, jnp.int32(0))


def kernel(x, glob, batch_ids):
    ids = batch_ids.astype(jnp.int32)
    return _broadcast_sc(ids, glob)


# speedup vs baseline: 1.4404x; 1.0377x over previous
import functools

import jax
import jax.numpy as jnp
from jax import lax
from jax.experimental import pallas as pl
from jax.experimental.pallas import tpu as pltpu
from jax.experimental.pallas import tpu_sc as plsc

N = 200000
B = 32
D = 256

NC = 2
NS = 16
NW = NC * NS

RPW0 = N // NW
NIDS = 6256
NVEC = NIDS // 16
RB = 32
NQ = 24

_mesh = plsc.VectorSubcoreMesh(core_axis_name="c", subcore_axis_name="s")


@functools.partial(
    pl.kernel,
    out_type=jax.ShapeDtypeStruct((N, D), jnp.float32),
    mesh=_mesh,
    scratch_types=[
        pltpu.VMEM((NIDS,), jnp.int32),
        pltpu.VMEM((B, D), jnp.float32),
        pltpu.VMEM((RB, D), jnp.float32),
        pltpu.VMEM((8, D), jnp.float32),
        pltpu.SemaphoreType.DMA,
    ],
)
def _broadcast_sc(ids_hbm, glob_hbm, out_hbm, idx_v, glob_v, buf, bblk, sem):
    wid = lax.axis_index("s") * NC + lax.axis_index("c")

    base = wid * RPW0 - ((2 * wid) & 7)
    nbase = (wid + 1) * RPW0 - ((2 * wid + 2) & 7)
    rpw = nbase - base

    pltpu.sync_copy(ids_hbm.at[pl.ds(pl.multiple_of(base, 8), NIDS)], idx_v)
    pltpu.sync_copy(glob_hbm, glob_v)

    def load_vec(m):
        return idx_v[pl.ds(pl.multiple_of(m * 16, 16), 16)]

    def lower_bound(t):
        tv = jnp.broadcast_to(t, (16,))
        pos = jnp.int32(0)
        for step in (256, 128, 64, 32, 16, 8, 4, 2, 1):
            cand = pos + step
            v = load_vec(jnp.minimum(cand - 1, NVEC - 1))
            ok = jnp.logical_and(cand <= NVEC, v[15] < t)
            pos = jnp.where(ok, cand, pos)
        v = load_vec(jnp.minimum(pos, NVEC - 1))
        w = jnp.where(v < tv, jnp.ones((16,), jnp.int32),
                      jnp.zeros((16,), jnp.int32))
        pc = w[0]
        for l in range(1, 16):
            pc = pc + w[l]
        return jnp.minimum(pos * 16 + pc, rpw)

    def emit(b, prev):
        nxt = lower_bound(b + 1)
        cnt = nxt - prev
        gs = base + prev
        ge = base + nxt
        iu = (gs + 7) & (-8)
        idn = ge & (-8)

        @pl.when(cnt > 0)
        def _():
            @pl.when(idn > iu)
            def _():
                def fill_row(r, c2):
                    for c in range(D // 16):
                        buf[r, pl.ds(c * 16, 16)] = glob_v[b, pl.ds(c * 16, 16)]
                    return c2

                lax.fori_loop(0, RB, fill_row, 0)

                size = idn - iu
                nfull = size // RB

                def dma_full(i, o):
                    pltpu.async_copy(
                        buf.at[pl.ds(0, RB)],
                        out_hbm.at[pl.ds(pl.multiple_of(o, 8), RB)],
                        sem)

                    @pl.when(i >= NQ)
                    def _():
                        pltpu.make_async_copy(
                            buf.at[pl.ds(0, RB)],
                            out_hbm.at[pl.ds(pl.multiple_of(iu, 8), RB)],
                            sem).wait()

                    return o + RB

                o = lax.fori_loop(0, nfull, dma_full, iu)

                for sz in [x for x in (128, 64, 32, 16, 8) if x < RB]:
                    @pl.when((size & sz) != 0)
                    def _(sz=sz, o=o):
                        pltpu.sync_copy(
                            buf.at[pl.ds(0, sz)],
                            out_hbm.at[pl.ds(pl.multiple_of(o, 8), sz)])
                    o = o + (size & sz)

                def drain(i, c2):
                    pltpu.make_async_copy(
                        buf.at[pl.ds(0, RB)],
                        out_hbm.at[pl.ds(pl.multiple_of(iu, 8), RB)],
                        sem).wait()
                    return c2

                lax.fori_loop(0, jnp.minimum(nfull, NQ), drain, 0)

            @pl.when((ge & 7) != 0)
            def _():
                p0 = idn - base
                vb = load_vec(p0 >> 4)
                sel_hi = (p0 & 15) == 8
                for r8 in range(8):
                    idr = jnp.where(sel_hi, vb[8 + r8], vb[r8])
                    for c in range(D // 16):
                        bblk[r8, pl.ds(c * 16, 16)] = glob_v[idr,
                                                             pl.ds(c * 16, 16)]
                pltpu.sync_copy(
                    bblk, out_hbm.at[pl.ds(pl.multiple_of(idn, 8), 8)])

        return nxt

    lax.fori_loop(0, B, emit, jnp.int32(0))


def kernel(x, glob, batch_ids):
    ids = batch_ids.astype(jnp.int32)
    return _broadcast_sc(ids, glob)


# speedup vs baseline: 1.4655x; 1.0174x over previous
import functools

import jax
import jax.numpy as jnp
from jax import lax
from jax.experimental import pallas as pl
from jax.experimental.pallas import tpu as pltpu
from jax.experimental.pallas import tpu_sc as plsc

N = 200000
B = 32
D = 256

NC = 2
NS = 16
NW = NC * NS

RPW0 = N // NW
NIDS = 6256
NVEC = NIDS // 16
RB = 16
NQ = 24

_mesh = plsc.VectorSubcoreMesh(core_axis_name="c", subcore_axis_name="s")


@functools.partial(
    pl.kernel,
    out_type=jax.ShapeDtypeStruct((N, D), jnp.float32),
    mesh=_mesh,
    scratch_types=[
        pltpu.VMEM((NIDS,), jnp.int32),
        pltpu.VMEM((B, D), jnp.float32),
        pltpu.VMEM((RB, D), jnp.float32),
        pltpu.VMEM((8, D), jnp.float32),
        pltpu.SemaphoreType.DMA,
    ],
)
def _broadcast_sc(ids_hbm, glob_hbm, out_hbm, idx_v, glob_v, buf, bblk, sem):
    wid = lax.axis_index("s") * NC + lax.axis_index("c")

    base = wid * RPW0 - ((2 * wid) & 7)
    nbase = (wid + 1) * RPW0 - ((2 * wid + 2) & 7)
    rpw = nbase - base

    pltpu.sync_copy(ids_hbm.at[pl.ds(pl.multiple_of(base, 8), NIDS)], idx_v)
    pltpu.sync_copy(glob_hbm, glob_v)

    def load_vec(m):
        return idx_v[pl.ds(pl.multiple_of(m * 16, 16), 16)]

    def lower_bound(t):
        tv = jnp.broadcast_to(t, (16,))
        pos = jnp.int32(0)
        for step in (256, 128, 64, 32, 16, 8, 4, 2, 1):
            cand = pos + step
            v = load_vec(jnp.minimum(cand - 1, NVEC - 1))
            ok = jnp.logical_and(cand <= NVEC, v[15] < t)
            pos = jnp.where(ok, cand, pos)
        v = load_vec(jnp.minimum(pos, NVEC - 1))
        w = jnp.where(v < tv, jnp.ones((16,), jnp.int32),
                      jnp.zeros((16,), jnp.int32))
        pc = w[0]
        for l in range(1, 16):
            pc = pc + w[l]
        return jnp.minimum(pos * 16 + pc, rpw)

    def emit(b, prev):
        nxt = lower_bound(b + 1)
        cnt = nxt - prev
        gs = base + prev
        ge = base + nxt
        iu = (gs + 7) & (-8)
        idn = ge & (-8)

        @pl.when(cnt > 0)
        def _():
            @pl.when(idn > iu)
            def _():
                def fill_row(r, c2):
                    for c in range(D // 16):
                        buf[r, pl.ds(c * 16, 16)] = glob_v[b, pl.ds(c * 16, 16)]
                    return c2

                lax.fori_loop(0, RB, fill_row, 0)

                size = idn - iu
                nfull = size // RB

                def dma_full(i, o):
                    pltpu.async_copy(
                        buf.at[pl.ds(0, RB)],
                        out_hbm.at[pl.ds(pl.multiple_of(o, 8), RB)],
                        sem)

                    @pl.when(i >= NQ)
                    def _():
                        pltpu.make_async_copy(
                            buf.at[pl.ds(0, RB)],
                            out_hbm.at[pl.ds(pl.multiple_of(iu, 8), RB)],
                            sem).wait()

                    return o + RB

                o = lax.fori_loop(0, nfull, dma_full, iu)

                for sz in [x for x in (128, 64, 32, 16, 8) if x < RB]:
                    @pl.when((size & sz) != 0)
                    def _(sz=sz, o=o):
                        pltpu.sync_copy(
                            buf.at[pl.ds(0, sz)],
                            out_hbm.at[pl.ds(pl.multiple_of(o, 8), sz)])
                    o = o + (size & sz)

                def drain(i, c2):
                    pltpu.make_async_copy(
                        buf.at[pl.ds(0, RB)],
                        out_hbm.at[pl.ds(pl.multiple_of(iu, 8), RB)],
                        sem).wait()
                    return c2

                lax.fori_loop(0, jnp.minimum(nfull, NQ), drain, 0)

            @pl.when((ge & 7) != 0)
            def _():
                p0 = idn - base
                vb = load_vec(p0 >> 4)
                sel_hi = (p0 & 15) == 8
                for r8 in range(8):
                    idr = jnp.where(sel_hi, vb[8 + r8], vb[r8])
                    for c in range(D // 16):
                        bblk[r8, pl.ds(c * 16, 16)] = glob_v[idr,
                                                             pl.ds(c * 16, 16)]
                pltpu.sync_copy(
                    bblk, out_hbm.at[pl.ds(pl.multiple_of(idn, 8), 8)])

        return nxt

    lax.fori_loop(0, B, emit, jnp.int32(0))


def kernel(x, glob, batch_ids):
    ids = batch_ids.astype(jnp.int32)
    return _broadcast_sc(ids, glob)


# speedup vs baseline: 1.4775x; 1.0082x over previous
import functools

import jax
import jax.numpy as jnp
from jax import lax
from jax.experimental import pallas as pl
from jax.experimental.pallas import tpu as pltpu
from jax.experimental.pallas import tpu_sc as plsc

N = 200000
B = 32
D = 256

NC = 2
NS = 16
NW = NC * NS

RPW0 = N // NW
NIDS = 6256
NVEC = NIDS // 16
RB = 8
NQ = 24

_mesh = plsc.VectorSubcoreMesh(core_axis_name="c", subcore_axis_name="s")


@functools.partial(
    pl.kernel,
    out_type=jax.ShapeDtypeStruct((N, D), jnp.float32),
    mesh=_mesh,
    scratch_types=[
        pltpu.VMEM((NIDS,), jnp.int32),
        pltpu.VMEM((B, D), jnp.float32),
        pltpu.VMEM((RB, D), jnp.float32),
        pltpu.VMEM((8, D), jnp.float32),
        pltpu.SemaphoreType.DMA,
    ],
)
def _broadcast_sc(ids_hbm, glob_hbm, out_hbm, idx_v, glob_v, buf, bblk, sem):
    wid = lax.axis_index("s") * NC + lax.axis_index("c")

    base = wid * RPW0 - ((2 * wid) & 7)
    nbase = (wid + 1) * RPW0 - ((2 * wid + 2) & 7)
    rpw = nbase - base

    pltpu.sync_copy(ids_hbm.at[pl.ds(pl.multiple_of(base, 8), NIDS)], idx_v)
    pltpu.sync_copy(glob_hbm, glob_v)

    def load_vec(m):
        return idx_v[pl.ds(pl.multiple_of(m * 16, 16), 16)]

    def lower_bound(t):
        tv = jnp.broadcast_to(t, (16,))
        pos = jnp.int32(0)
        for step in (256, 128, 64, 32, 16, 8, 4, 2, 1):
            cand = pos + step
            v = load_vec(jnp.minimum(cand - 1, NVEC - 1))
            ok = jnp.logical_and(cand <= NVEC, v[15] < t)
            pos = jnp.where(ok, cand, pos)
        v = load_vec(jnp.minimum(pos, NVEC - 1))
        w = jnp.where(v < tv, jnp.ones((16,), jnp.int32),
                      jnp.zeros((16,), jnp.int32))
        pc = w[0]
        for l in range(1, 16):
            pc = pc + w[l]
        return jnp.minimum(pos * 16 + pc, rpw)

    def emit(b, prev):
        nxt = lower_bound(b + 1)
        cnt = nxt - prev
        gs = base + prev
        ge = base + nxt
        iu = (gs + 7) & (-8)
        idn = ge & (-8)

        @pl.when(cnt > 0)
        def _():
            @pl.when(idn > iu)
            def _():
                def fill_row(r, c2):
                    for c in range(D // 16):
                        buf[r, pl.ds(c * 16, 16)] = glob_v[b, pl.ds(c * 16, 16)]
                    return c2

                lax.fori_loop(0, RB, fill_row, 0)

                size = idn - iu
                nfull = size // RB

                def dma_full(i, o):
                    pltpu.async_copy(
                        buf.at[pl.ds(0, RB)],
                        out_hbm.at[pl.ds(pl.multiple_of(o, 8), RB)],
                        sem)

                    @pl.when(i >= NQ)
                    def _():
                        pltpu.make_async_copy(
                            buf.at[pl.ds(0, RB)],
                            out_hbm.at[pl.ds(pl.multiple_of(iu, 8), RB)],
                            sem).wait()

                    return o + RB

                o = lax.fori_loop(0, nfull, dma_full, iu)

                for sz in [x for x in (128, 64, 32, 16, 8) if x < RB]:
                    @pl.when((size & sz) != 0)
                    def _(sz=sz, o=o):
                        pltpu.sync_copy(
                            buf.at[pl.ds(0, sz)],
                            out_hbm.at[pl.ds(pl.multiple_of(o, 8), sz)])
                    o = o + (size & sz)

                def drain(i, c2):
                    pltpu.make_async_copy(
                        buf.at[pl.ds(0, RB)],
                        out_hbm.at[pl.ds(pl.multiple_of(iu, 8), RB)],
                        sem).wait()
                    return c2

                lax.fori_loop(0, jnp.minimum(nfull, NQ), drain, 0)

            @pl.when((ge & 7) != 0)
            def _():
                p0 = idn - base
                vb = load_vec(p0 >> 4)
                sel_hi = (p0 & 15) == 8
                for r8 in range(8):
                    idr = jnp.where(sel_hi, vb[8 + r8], vb[r8])
                    for c in range(D // 16):
                        bblk[r8, pl.ds(c * 16, 16)] = glob_v[idr,
                                                             pl.ds(c * 16, 16)]
                pltpu.sync_copy(
                    bblk, out_hbm.at[pl.ds(pl.multiple_of(idn, 8), 8)])

        return nxt

    lax.fori_loop(0, B, emit, jnp.int32(0))


def kernel(x, glob, batch_ids):
    ids = batch_ids.astype(jnp.int32)
    return _broadcast_sc(ids, glob)


# speedup vs baseline: 1.4832x; 1.0039x over previous
import functools

import jax
import jax.numpy as jnp
from jax import lax
from jax.experimental import pallas as pl
from jax.experimental.pallas import tpu as pltpu
from jax.experimental.pallas import tpu_sc as plsc

N = 200000
B = 32
D = 256

NC = 2
NS = 16
NW = NC * NS

RPW0 = N // NW
NIDS = 6256
NVEC = NIDS // 16
RB = 8
NQ = 48

_mesh = plsc.VectorSubcoreMesh(core_axis_name="c", subcore_axis_name="s")


@functools.partial(
    pl.kernel,
    out_type=jax.ShapeDtypeStruct((N, D), jnp.float32),
    mesh=_mesh,
    scratch_types=[
        pltpu.VMEM((NIDS,), jnp.int32),
        pltpu.VMEM((B, D), jnp.float32),
        pltpu.VMEM((RB, D), jnp.float32),
        pltpu.VMEM((8, D), jnp.float32),
        pltpu.SemaphoreType.DMA,
    ],
)
def _broadcast_sc(ids_hbm, glob_hbm, out_hbm, idx_v, glob_v, buf, bblk, sem):
    wid = lax.axis_index("s") * NC + lax.axis_index("c")

    base = wid * RPW0 - ((2 * wid) & 7)
    nbase = (wid + 1) * RPW0 - ((2 * wid + 2) & 7)
    rpw = nbase - base

    pltpu.sync_copy(ids_hbm.at[pl.ds(pl.multiple_of(base, 8), NIDS)], idx_v)
    pltpu.sync_copy(glob_hbm, glob_v)

    def load_vec(m):
        return idx_v[pl.ds(pl.multiple_of(m * 16, 16), 16)]

    def lower_bound(t):
        tv = jnp.broadcast_to(t, (16,))
        pos = jnp.int32(0)
        for step in (256, 128, 64, 32, 16, 8, 4, 2, 1):
            cand = pos + step
            v = load_vec(jnp.minimum(cand - 1, NVEC - 1))
            ok = jnp.logical_and(cand <= NVEC, v[15] < t)
            pos = jnp.where(ok, cand, pos)
        v = load_vec(jnp.minimum(pos, NVEC - 1))
        w = jnp.where(v < tv, jnp.ones((16,), jnp.int32),
                      jnp.zeros((16,), jnp.int32))
        pc = w[0]
        for l in range(1, 16):
            pc = pc + w[l]
        return jnp.minimum(pos * 16 + pc, rpw)

    def emit(b, prev):
        nxt = lower_bound(b + 1)
        cnt = nxt - prev
        gs = base + prev
        ge = base + nxt
        iu = (gs + 7) & (-8)
        idn = ge & (-8)

        @pl.when(cnt > 0)
        def _():
            @pl.when(idn > iu)
            def _():
                def fill_row(r, c2):
                    for c in range(D // 16):
                        buf[r, pl.ds(c * 16, 16)] = glob_v[b, pl.ds(c * 16, 16)]
                    return c2

                lax.fori_loop(0, RB, fill_row, 0)

                size = idn - iu
                nfull = size // RB

                def dma_full(i, o):
                    pltpu.async_copy(
                        buf.at[pl.ds(0, RB)],
                        out_hbm.at[pl.ds(pl.multiple_of(o, 8), RB)],
                        sem)

                    @pl.when(i >= NQ)
                    def _():
                        pltpu.make_async_copy(
                            buf.at[pl.ds(0, RB)],
                            out_hbm.at[pl.ds(pl.multiple_of(iu, 8), RB)],
                            sem).wait()

                    return o + RB

                o = lax.fori_loop(0, nfull, dma_full, iu)

                for sz in [x for x in (128, 64, 32, 16, 8) if x < RB]:
                    @pl.when((size & sz) != 0)
                    def _(sz=sz, o=o):
                        pltpu.sync_copy(
                            buf.at[pl.ds(0, sz)],
                            out_hbm.at[pl.ds(pl.multiple_of(o, 8), sz)])
                    o = o + (size & sz)

                def drain(i, c2):
                    pltpu.make_async_copy(
                        buf.at[pl.ds(0, RB)],
                        out_hbm.at[pl.ds(pl.multiple_of(iu, 8), RB)],
                        sem).wait()
                    return c2

                lax.fori_loop(0, jnp.minimum(nfull, NQ), drain, 0)

            @pl.when((ge & 7) != 0)
            def _():
                p0 = idn - base
                vb = load_vec(p0 >> 4)
                sel_hi = (p0 & 15) == 8
                for r8 in range(8):
                    idr = jnp.where(sel_hi, vb[8 + r8], vb[r8])
                    for c in range(D // 16):
                        bblk[r8, pl.ds(c * 16, 16)] = glob_v[idr,
                                                             pl.ds(c * 16, 16)]
                pltpu.sync_copy(
                    bblk, out_hbm.at[pl.ds(pl.multiple_of(idn, 8), 8)])

        return nxt

    lax.fori_loop(0, B, emit, jnp.int32(0))


def kernel(x, glob, batch_ids):
    ids = batch_ids.astype(jnp.int32)
    return _broadcast_sc(ids, glob)
